# Initial kernel scaffold; baseline (speedup 1.0000x reference)
import jax
import jax.numpy as jnp
from jax.experimental import pallas as pl


def kernel(x0, x1, x2, x3, boxes, seq_len):
    raise NotImplementedError("write your pallas kernel here")


# speedup vs baseline: 53.8136x; 53.8136x over previous
import functools

import jax
import jax.numpy as jnp
from jax import lax
from jax.experimental import pallas as pl
from jax.experimental.pallas import tpu as pltpu
from jax.experimental.pallas import tpu_sc as plsc

B = 4
C = 256
N = 250
R = B * N
OUT = 32
MAXG = 8
MAXK = OUT * MAXG
NW = 32
TPW = (R + NW - 1) // NW
LW = (2048, 1024, 512, 256)
SCALES = (0.25, 0.125, 0.0625, 0.03125)
BUCKETS = (48, 80, 144, 272)
WINW = BUCKETS[-1]
NCH = C // 16

_T3 = float(128.0 * 2.0 ** (-1e-6))
_T4 = float(256.0 * 2.0 ** (-1e-6))
_T5 = float(512.0 * 2.0 ** (-1e-6))


def _sload(ref, i):
    return ref[pl.ds(i, 16)][0]


def _sel_by(i, vals, dtype):
    acc = jnp.full((16,), vals[-1], dtype=dtype)
    for j in range(len(vals) - 2, -1, -1):
        acc = jnp.where(i == j, jnp.asarray(vals[j], dtype=dtype), acc)
    return acc


def _body(x0t, x1t, x2t, x3t, bx1, bx2, out,
          bv1, bv2, m_bidx, m_p0, m_g, m_li, m_bk, m_wdm1,
          m_start, m_binw, m_delta, m_invg, m_wdf,
          s_lo, s_hi, s_wlo, s_whi, win, outv):
    wid = lax.axis_index("s") * 2 + lax.axis_index("c")
    xts = (x0t, x1t, x2t, x3t)

    pltpu.sync_copy(bx1, bv1)
    pltpu.sync_copy(bx2, bv2)

    iota = lax.iota(jnp.int32, 16)
    iotaf = iota.astype(jnp.float32)

    for ch in range(2):
        t = ch * 16 + iota
        rv = t * NW + wid
        rvc = jnp.minimum(rv, R - 1)
        x1 = bv1[pl.ds(wid * 32 + ch * 16, 16)]
        x2 = bv2[pl.ds(wid * 32 + ch * 16, 16)]
        s = x2 - x1
        one = jnp.full((16,), 1, jnp.int32)
        zero = jnp.full((16,), 0, jnp.int32)
        li = (jnp.where(s >= _T3, one, zero) + jnp.where(s >= _T4, one, zero)
              + jnp.where(s >= _T5, one, zero))
        wdf = _sel_by(li, [float(w) for w in LW], jnp.float32)
        sc = _sel_by(li, list(SCALES), jnp.float32)
        start = x1 * sc - 0.5
        roiw = s * sc
        binw = roiw * (1.0 / OUT)
        gi = binw.astype(jnp.int32)
        g = gi + jnp.where(gi.astype(jnp.float32) < binw, one, zero)
        g = jnp.maximum(g, 1)
        gf = g.astype(jnp.float32)
        delta = binw / gf
        invg = 1.0 / gf
        bk = (jnp.where(roiw > 36.0, one, zero) + jnp.where(roiw > 68.0, one, zero)
              + jnp.where(roiw > 132.0, one, zero))
        bsz = _sel_by(bk, list(BUCKETS), jnp.int32)
        wdi = _sel_by(li, list(LW), jnp.int32)
        p0 = jnp.maximum(start, 0.0).astype(jnp.int32)
        p0 = jnp.bitwise_and(p0, -8)
        p0 = jnp.maximum(0, jnp.minimum(p0, wdi - bsz))
        bidx = (jnp.where(rvc >= N, one, zero) + jnp.where(rvc >= 2 * N, one, zero)
                + jnp.where(rvc >= 3 * N, one, zero))
        sl = pl.ds(ch * 16, 16)
        m_bidx[sl] = bidx
        m_p0[sl] = p0
        m_g[sl] = g
        m_li[sl] = li
        m_bk[sl] = bk
        m_wdm1[sl] = wdi - 1
        m_start[sl] = start
        m_binw[sl] = binw
        m_delta[sl] = delta
        m_invg[sl] = invg
        m_wdf[sl] = wdf

    nr = lax.shift_right_logical(R + NW - 1 - wid, 5)

    def roi_body(t, carry):
        r = t * NW + wid
        bidx = _sload(m_bidx, t)
        p0 = pl.multiple_of(_sload(m_p0, t), 8)
        g = _sload(m_g, t)
        li = _sload(m_li, t)
        bk = _sload(m_bk, t)
        wdm1 = _sload(m_wdm1, t)
        start = _sload(m_start, t)
        binw = _sload(m_binw, t)
        delta = _sload(m_delta, t)
        invg = _sload(m_invg, t)
        wdf = _sload(m_wdf, t)

        for lv in range(4):
            @pl.when(jnp.logical_and(li == lv, bk == 0))
            def _(lv=lv):
                pltpu.sync_copy(xts[lv].at[bidx, pl.ds(p0, BUCKETS[0]), :],
                                win.at[pl.ds(0, BUCKETS[0]), :])
        for bj in range(1, 4):
            @pl.when(bk == bj)
            def _(bj=bj):
                pltpu.sync_copy(x3t.at[bidx, pl.ds(p0, BUCKETS[bj]), :],
                                win.at[pl.ds(0, BUCKETS[bj]), :])

        startv = jnp.full((16,), start, jnp.float32)
        binwv = jnp.full((16,), binw, jnp.float32)
        deltav = jnp.full((16,), delta, jnp.float32)
        invgv = jnp.full((16,), invg, jnp.float32)
        wdfv = jnp.full((16,), wdf, jnp.float32)
        gv = jnp.full((16,), g, jnp.int32)
        gfv = gv.astype(jnp.float32)
        wdm1v = jnp.full((16,), wdm1, jnp.int32)
        p0v = jnp.full((16,), p0, jnp.int32)

        def samp_body(cc, scarry):
            kk = cc * 16 + iota
            kkf = kk.astype(jnp.float32)
            bb = (kkf / gfv).astype(jnp.int32)
            ii = kk - bb * gv
            xs = startv + bb.astype(jnp.float32) * binwv \
                + (ii.astype(jnp.float32) + 0.5) * deltav
            validm = jnp.logical_and(xs >= -1.0, xs <= wdfv)
            xc = jnp.maximum(xs, 0.0)
            xl0 = xc.astype(jnp.int32)
            xl0f = xl0.astype(jnp.float32)
            hic = xl0f >= wdfv - 1.0
            lo = jnp.where(hic, wdm1v, xl0)
            hi = jnp.where(hic, wdm1v, xl0 + 1)
            lx = jnp.where(hic, 0.0, xc - xl0f)
            wlo = jnp.where(validm, (1.0 - lx) * invgv, 0.0)
            whi = jnp.where(validm, lx * invgv, 0.0)
            sl = pl.ds(cc * 16, 16)
            s_lo[sl] = lo - p0v
            s_hi[sl] = hi - p0v
            s_wlo[sl] = wlo
            s_whi[sl] = whi
            return scarry

        lax.fori_loop(0, 2 * g, samp_body, 0, unroll=False)

        def bin_body(b, bcarry):
            k0 = b * g

            def samp_acc(i, acc):
                k = k0 + i
                lo = _sload(s_lo, k)
                hi = _sload(s_hi, k)
                wlo_v = jnp.full((16,), _sload(s_wlo, k), jnp.float32)
                whi_v = jnp.full((16,), _sload(s_whi, k), jnp.float32)
                return tuple(
                    acc[c] + wlo_v * win[lo, pl.ds(c * 16, 16)]
                    + whi_v * win[hi, pl.ds(c * 16, 16)]
                    for c in range(NCH))

            zero = jnp.zeros((16,), jnp.float32)
            acc = lax.fori_loop(0, g, samp_acc, (zero,) * NCH, unroll=False)
            for c in range(NCH):
                outv[pl.ds(b * C + c * 16, 16)] = acc[c]
            return bcarry

        lax.fori_loop(0, OUT, bin_body, 0, unroll=False)

        pltpu.sync_copy(outv, out.at[r])
        return carry

    lax.fori_loop(0, nr, roi_body, 0, unroll=False)


@functools.partial(jax.jit, static_argnums=(5,))
def _run(x0, x1, x2, x3, boxes, _seq_len_static):
    x0t = jnp.transpose(x0, (0, 2, 1))
    x1t = jnp.transpose(x1, (0, 2, 1))
    x2t = jnp.transpose(x2, (0, 2, 1))
    x3t = jnp.pad(jnp.transpose(x3, (0, 2, 1)), ((0, 0), (0, WINW - LW[3]), (0, 0)))
    pos = jnp.arange(NW * TPW, dtype=jnp.int32)
    ridx = jnp.minimum((pos % NW) * NW + pos // NW, R - 1)
    bx = boxes.reshape(R, 2)[ridx]
    bx1 = bx[:, 0]
    bx2 = bx[:, 1]
    scratch = [
        pltpu.VMEM((NW * TPW,), jnp.float32),
        pltpu.VMEM((NW * TPW,), jnp.float32),
        pltpu.VMEM((NW + 16,), jnp.int32),
        pltpu.VMEM((NW + 16,), jnp.int32),
        pltpu.VMEM((NW + 16,), jnp.int32),
        pltpu.VMEM((NW + 16,), jnp.int32),
        pltpu.VMEM((NW + 16,), jnp.int32),
        pltpu.VMEM((NW + 16,), jnp.int32),
        pltpu.VMEM((NW + 16,), jnp.float32),
        pltpu.VMEM((NW + 16,), jnp.float32),
        pltpu.VMEM((NW + 16,), jnp.float32),
        pltpu.VMEM((NW + 16,), jnp.float32),
        pltpu.VMEM((NW + 16,), jnp.float32),
        pltpu.VMEM((MAXK + 16,), jnp.int32),
        pltpu.VMEM((MAXK + 16,), jnp.int32),
        pltpu.VMEM((MAXK + 16,), jnp.float32),
        pltpu.VMEM((MAXK + 16,), jnp.float32),
        pltpu.VMEM((WINW, C), jnp.float32),
        pltpu.VMEM((OUT * C,), jnp.float32),
    ]
    fn = pl.kernel(
        _body,
        out_type=jax.ShapeDtypeStruct((R, OUT * C), jnp.float32),
        mesh=plsc.VectorSubcoreMesh(core_axis_name="c", subcore_axis_name="s"),
        scratch_types=scratch,
    )
    o = fn(x0t, x1t, x2t, x3t, bx1, bx2)
    return jnp.transpose(o.reshape(R, OUT, C), (0, 2, 1))


def kernel(x0, x1, x2, x3, boxes, seq_len):
    return _run(x0, x1, x2, x3, boxes, 8192)
